# baseline (speedup 1.0000x reference)
import jax
import jax.numpy as jnp
from jax.experimental import pallas as pl
from jax.experimental.pallas import tpu as pltpu


def _pooler_kernel(h_ref, m_ref, w_ref, b_ref, out_ref):
    m = (m_ref[:, :, 3:4] != 0).astype(jnp.float32)
    h = h_ref[...]
    pooled_sum = jnp.sum(h * m, axis=1)
    denom = jnp.maximum(jnp.sum(m, axis=1), 1e-7)
    pooled = pooled_sum / denom
    proj = jnp.dot(pooled, w_ref[...], preferred_element_type=jnp.float32)
    out_ref[...] = jnp.tanh(proj + b_ref[...])


def kernel(hidden, token_mask, weight, bias):
    B, S, H = hidden.shape
    O = weight.shape[1]
    b_tile = 8
    grid = (B // b_tile,)

    mask_bytes = jax.lax.bitcast_convert_type(
        token_mask.astype(jnp.float32), jnp.uint8).reshape(B, S, 4 * H)

    return pl.pallas_call(
        _pooler_kernel,
        out_shape=jax.ShapeDtypeStruct((B, O), jnp.float32),
        grid=grid,
        in_specs=[
            pl.BlockSpec((b_tile, S, H), lambda b: (b, 0, 0)),
            pl.BlockSpec((b_tile, S, 128), lambda b: (b, 0, 0)),
            pl.BlockSpec((H, O), lambda b: (0, 0)),
            pl.BlockSpec((1, O), lambda b: (0, 0)),
        ],
        out_specs=pl.BlockSpec((b_tile, O), lambda b: (b, 0)),
        compiler_params=pltpu.CompilerParams(
            dimension_semantics=("arbitrary",),
            vmem_limit_bytes=64 * 1024 * 1024),
        cost_estimate=pl.CostEstimate(
            flops=3 * B * S * H + 2 * B * H * O,
            transcendentals=B * O,
            bytes_accessed=int(hidden.nbytes + B * S * 128
                               + weight.nbytes + B * O * 4)),
    )(hidden.astype(jnp.float32),
      mask_bytes,
      weight.astype(jnp.float32),
      bias.astype(jnp.float32).reshape(1, O))


# speedup vs baseline: 20.9561x; 20.9561x over previous
import jax
import jax.numpy as jnp
from jax.experimental import pallas as pl
from jax.experimental.pallas import tpu as pltpu


def _pooler_kernel(h_ref, m_ref, w_ref, b_ref, out_ref):
    m = m_ref[:, :, 0:1]
    h = h_ref[...]
    pooled_sum = jnp.sum(h * m, axis=1)
    denom = jnp.maximum(jnp.sum(m, axis=1), 1e-7)
    pooled = pooled_sum / denom
    proj = jnp.dot(pooled, w_ref[...], preferred_element_type=jnp.float32)
    out_ref[...] = jnp.tanh(proj + b_ref[...])


def kernel(hidden, token_mask, weight, bias):
    B, S, H = hidden.shape
    O = weight.shape[1]
    b_tile = 8
    grid = (B // b_tile,)

    return pl.pallas_call(
        _pooler_kernel,
        out_shape=jax.ShapeDtypeStruct((B, O), jnp.float32),
        grid=grid,
        in_specs=[
            pl.BlockSpec((b_tile, S, H), lambda b: (b, 0, 0)),
            pl.BlockSpec((b_tile, S, 128), lambda b: (b, 0, 0)),
            pl.BlockSpec((H, O), lambda b: (0, 0)),
            pl.BlockSpec((1, O), lambda b: (0, 0)),
        ],
        out_specs=pl.BlockSpec((b_tile, O), lambda b: (b, 0)),
        compiler_params=pltpu.CompilerParams(
            dimension_semantics=("arbitrary",),
            vmem_limit_bytes=64 * 1024 * 1024),
        cost_estimate=pl.CostEstimate(
            flops=3 * B * S * H + 2 * B * H * O,
            transcendentals=B * O,
            bytes_accessed=int(hidden.nbytes + hidden.nbytes // 6
                               + weight.nbytes + B * O * 4)),
    )(hidden.astype(jnp.float32),
      token_mask.astype(jnp.float32),
      weight.astype(jnp.float32),
      bias.astype(jnp.float32).reshape(1, O))
